# baseline (speedup 1.0000x reference)
import jax
import jax.numpy as jnp
from jax import lax
from jax.experimental import pallas as pl
from jax.experimental.pallas import tpu as pltpu
from jax.experimental.pallas import tpu_sc as plsc

_CHUNK_ROWS = 32


def kernel(x, enc_weight):
    seq_len = x.shape[1]
    d = enc_weight.shape[1]
    mesh = plsc.VectorSubcoreMesh(core_axis_name="c", subcore_axis_name="s")
    num_workers = mesh.num_cores * mesh.num_subcores
    rows_per_worker = seq_len // num_workers
    chunks = rows_per_worker // _CHUNK_ROWS

    def body(w_hbm, o_hbm, buf0, buf1, si0, si1, so0, so1):
        wid = lax.axis_index("s") * mesh.num_cores + lax.axis_index("c")
        base = wid * rows_per_worker
        bufs = (buf0, buf1)
        in_sems = (si0, si1)
        out_sems = (so0, so1)

        in_flight = [None, None]
        for i in range(chunks):
            b = i % 2
            start = base + i * _CHUNK_ROWS
            if in_flight[b] is not None:
                in_flight[b].wait()
            cp_in = pltpu.make_async_copy(
                w_hbm.at[pl.ds(start, _CHUNK_ROWS)], bufs[b], in_sems[b])
            cp_in.start()
            cp_in.wait()
            cp_out = pltpu.make_async_copy(
                bufs[b], o_hbm.at[pl.ds(start, _CHUNK_ROWS)], out_sems[b])
            cp_out.start()
            in_flight[b] = cp_out
        for cp in in_flight:
            if cp is not None:
                cp.wait()

    return pl.kernel(
        body,
        out_type=jax.ShapeDtypeStruct((seq_len, d), enc_weight.dtype),
        mesh=mesh,
        scratch_types=[
            pltpu.VMEM((_CHUNK_ROWS, d), enc_weight.dtype),
            pltpu.VMEM((_CHUNK_ROWS, d), enc_weight.dtype),
            pltpu.SemaphoreType.DMA,
            pltpu.SemaphoreType.DMA,
            pltpu.SemaphoreType.DMA,
            pltpu.SemaphoreType.DMA,
        ],
    )(enc_weight)
